# baseline (speedup 1.0000x reference)
import functools

import jax
import jax.numpy as jnp
from jax.experimental import pallas as pl
from jax.experimental.pallas import tpu as pltpu

_NEG = float('-inf')
_NUM_EXTRACT = 63
_ROWS_PER_STEP = 8


def _row_topk(logits_ref, width, ngroups):
  z2 = logits_ref[0]
  nsg = 8 * ngroups

  sg = jax.lax.broadcasted_iota(jnp.int32, (nsg, 128), 0)
  l = jax.lax.broadcasted_iota(jnp.int32, (nsg, 128), 1)
  colidx = (sg // ngroups) * width + (sg % ngroups) * 128 + l
  m0 = jnp.max(z2)
  amax = jnp.min(jnp.where(z2 == m0, colidx, jnp.int32(2**30)))

  g0 = jnp.max(z2.reshape(8, ngroups, 128), axis=2)
  sub2 = jax.lax.broadcasted_iota(jnp.int32, (8, ngroups), 0)
  grp2 = jax.lax.broadcasted_iota(jnp.int32, (8, ngroups), 1)
  flat2 = sub2 * ngroups + grp2
  lane = jax.lax.broadcasted_iota(jnp.int32, (1, 128), 1)

  def body(i, carry):
    m, g_tab, vals, cnts = carry
    pos = jnp.min(jnp.where(g_tab == m, flat2, jnp.int32(2**30)))
    zg = logits_ref[0, pos, :]
    cnt = jnp.sum(jnp.where(zg == m, 1.0, 0.0))
    repl = jnp.max(jnp.where(zg < m, zg, _NEG))
    g_tab = jnp.where(flat2 == pos, repl, g_tab)
    vals = jnp.where(lane == i, m, vals)
    cnts = jnp.where(lane == i, cnt, cnts)
    return jnp.max(g_tab), g_tab, vals, cnts

  vals0 = jnp.full((1, 128), _NEG, dtype=jnp.float32)
  cnts0 = jnp.zeros((1, 128), dtype=jnp.float32)
  _, _, vals, cnts = jax.lax.fori_loop(
      0, _NUM_EXTRACT, body, (m0, g0, vals0, cnts0))
  return amax, m0, vals, cnts


def _score_from_topk(m0, vals, cnts, temp, k, p, tri, lane):
  e = jnp.exp((vals - m0) / temp)
  ce = cnts * e
  cumcnt = jax.lax.dot(cnts, tri, precision=jax.lax.Precision.HIGHEST)
  cumce = jax.lax.dot(ce, tri, precision=jax.lax.Precision.HIGHEST)

  kf = jnp.clip(k, 1, _NUM_EXTRACT).astype(jnp.float32)
  jstar = jnp.min(jnp.where(cumcnt >= kf, lane, jnp.int32(999)))
  tau = jnp.max(jnp.where(lane == jstar, vals, _NEG))
  pmask = vals >= tau

  e_tot = jnp.sum(jnp.where(pmask, ce, 0.0))
  thr = p * e_tot
  c_before = cumce - ce
  epos = e > 0.0
  kept = jnp.floor((thr - c_before) / jnp.where(epos, e, 1.0)) + 1.0
  kept = jnp.clip(kept, 0.0, cnts)
  kept = jnp.where(pmask & epos, kept, 0.0)
  return 1.0 / jnp.sum(kept * e)


def _sampler_kernel(rows_ref, temps_ref, ks_ref, ps_ref, *refs, width,
                    ngroups):
  logit_refs = refs[:_ROWS_PER_STEP]
  score_ref, samp_ref = refs[_ROWS_PER_STEP:]
  i = pl.program_id(0)

  tri = (jax.lax.broadcasted_iota(jnp.int32, (128, 128), 0)
         <= jax.lax.broadcasted_iota(jnp.int32, (128, 128), 1)).astype(
             jnp.float32)
  lane = jax.lax.broadcasted_iota(jnp.int32, (1, 128), 1)
  sub8 = jax.lax.broadcasted_iota(jnp.int32, (_ROWS_PER_STEP, 128), 0)

  sc_acc = jnp.zeros((_ROWS_PER_STEP, 128), jnp.float32)
  tok_acc = jnp.zeros((_ROWS_PER_STEP, 128), jnp.int32)
  for j in range(_ROWS_PER_STEP):
    t = i * _ROWS_PER_STEP + j
    amax, m0, vals, cnts = _row_topk(logit_refs[j], width, ngroups)
    score = _score_from_topk(m0, vals, cnts, temps_ref[t], ks_ref[t],
                             ps_ref[t], tri, lane)
    sc_acc = jnp.where(sub8 == j, score, sc_acc)
    tok_acc = jnp.where(sub8 == j, amax, tok_acc)

  score_ref[0] = sc_acc
  samp_ref[0] = tok_acc


def kernel(input_logits, cu_seqlens_q, relative_idx, batch_offsets,
           cu_filtered, temperatures, num_transfer, top_k, top_p):
  del batch_offsets, num_transfer
  rows, v = input_logits.shape
  t_total = relative_idx.shape[0]
  nb = cu_filtered.shape[0] - 1
  width = v // 8
  width_pad = ((width + 127) // 128) * 128
  ngroups = width_pad // 128
  nsteps = t_total // _ROWS_PER_STEP

  counts = jnp.diff(cu_filtered)
  group_ids = jnp.repeat(jnp.arange(nb), counts, total_repeat_length=t_total)
  global_rows = (jnp.take(cu_seqlens_q[:-1], group_ids, axis=0)
                 + relative_idx).astype(jnp.int32)

  logits3 = input_logits.reshape(rows, 8, width)
  logits3 = jnp.pad(logits3, ((0, 0), (0, 0), (0, width_pad - width)),
                    constant_values=_NEG)
  logits3 = logits3.reshape(rows, 8 * ngroups, 128)

  def in_map(j):
    return lambda i, rows_r, temps_r, ks_r, ps_r: (
        rows_r[i * _ROWS_PER_STEP + j], 0, 0)

  out_map = lambda i, rows_r, temps_r, ks_r, ps_r: (i, 0, 0)

  grid_spec = pltpu.PrefetchScalarGridSpec(
      num_scalar_prefetch=4,
      grid=(nsteps,),
      in_specs=[pl.BlockSpec((1, 8 * ngroups, 128), in_map(j))
                for j in range(_ROWS_PER_STEP)],
      out_specs=[
          pl.BlockSpec((1, _ROWS_PER_STEP, 128), out_map),
          pl.BlockSpec((1, _ROWS_PER_STEP, 128), out_map),
      ],
  )

  score3, samp3 = pl.pallas_call(
      functools.partial(_sampler_kernel, width=width, ngroups=ngroups),
      grid_spec=grid_spec,
      out_shape=[
          jax.ShapeDtypeStruct((nsteps, _ROWS_PER_STEP, 128), jnp.float32),
          jax.ShapeDtypeStruct((nsteps, _ROWS_PER_STEP, 128), jnp.int32),
      ],
  )(global_rows, temperatures, top_k, top_p,
    *([logits3] * _ROWS_PER_STEP))

  return (samp3[:, :, 0].reshape(t_total),
          score3[:, :, 0].reshape(t_total))


# speedup vs baseline: 3.5338x; 3.5338x over previous
import functools

import jax
import jax.numpy as jnp
from jax.experimental import pallas as pl
from jax.experimental.pallas import tpu as pltpu

_NEG = float('-inf')
_NUM_EXTRACT = 63
_RPS = 8


def _sampler_kernel(rows_ref, temps_ref, ks_ref, ps_ref, *refs, width):
  logit_refs = refs[:_RPS]
  score_ref, samp_ref = refs[_RPS:]
  i = pl.program_id(0)

  lane = jax.lax.broadcasted_iota(jnp.int32, (1, 128), 1)
  sub8 = jax.lax.broadcasted_iota(jnp.int32, (_RPS, 128), 0)

  zs = [logit_refs[j][0] for j in range(_RPS)]

  shp = zs[0].shape
  sub = jax.lax.broadcasted_iota(jnp.int32, shp, 0)
  lanec = jax.lax.broadcasted_iota(jnp.int32, shp, 1)
  colidx = sub * width + lanec
  m0s = [jnp.max(z) for z in zs]
  amaxs = [jnp.min(jnp.where(zs[j] == m0s[j], colidx, jnp.int32(2**30)))
           for j in range(_RPS)]

  def body(it, carry):
    ms, vals, ccum = carry
    new_ms = []
    for j in range(_RPS):
      ge = zs[j] >= ms[j]
      c = jnp.sum(jnp.where(ge, 1.0, 0.0))
      vals = jnp.where((sub8 == j) & (lane == it), ms[j], vals)
      ccum = jnp.where((sub8 == j) & (lane == it), c, ccum)
      new_ms.append(jnp.max(jnp.where(ge, _NEG, zs[j])))
    return tuple(new_ms), vals, ccum

  vals0 = jnp.full((_RPS, 128), _NEG, dtype=jnp.float32)
  ccum0 = jnp.zeros((_RPS, 128), dtype=jnp.float32)
  _, vals, cumcnt = jax.lax.fori_loop(
      0, _NUM_EXTRACT, body, (tuple(m0s), vals0, ccum0))

  tvec = jnp.zeros((_RPS, 1), jnp.float32)
  kvec = jnp.zeros((_RPS, 1), jnp.float32)
  pvec = jnp.zeros((_RPS, 1), jnp.float32)
  m0vec = jnp.zeros((_RPS, 1), jnp.float32)
  sub81 = jax.lax.broadcasted_iota(jnp.int32, (_RPS, 1), 0)
  for j in range(_RPS):
    t = i * _RPS + j
    tvec = jnp.where(sub81 == j, temps_ref[t], tvec)
    kf = jnp.clip(ks_ref[t], 1, _NUM_EXTRACT).astype(jnp.float32)
    kvec = jnp.where(sub81 == j, kf, kvec)
    pvec = jnp.where(sub81 == j, ps_ref[t], pvec)
    m0vec = jnp.where(sub81 == j, m0s[j], m0vec)

  shift = (jax.lax.broadcasted_iota(jnp.int32, (128, 128), 0) + 1
           == jax.lax.broadcasted_iota(jnp.int32, (128, 128), 1)).astype(
               jnp.float32)
  cnts = cumcnt - jax.lax.dot(cumcnt, shift,
                              precision=jax.lax.Precision.HIGHEST)

  e = jnp.exp((vals - m0vec) / tvec)
  ce = cnts * e
  tri = (jax.lax.broadcasted_iota(jnp.int32, (128, 128), 0)
         <= jax.lax.broadcasted_iota(jnp.int32, (128, 128), 1)).astype(
             jnp.float32)
  cumce = jax.lax.dot(ce, tri, precision=jax.lax.Precision.HIGHEST)

  jstar = jnp.min(jnp.where(cumcnt >= kvec, lane, jnp.int32(999)),
                  axis=1, keepdims=True)
  tau = jnp.max(jnp.where(lane == jstar, vals, _NEG), axis=1, keepdims=True)
  pmask = vals >= tau

  e_tot = jnp.sum(jnp.where(pmask, ce, 0.0), axis=1, keepdims=True)
  thr = pvec * e_tot
  c_before = cumce - ce
  epos = e > 0.0
  kept = jnp.floor((thr - c_before) / jnp.where(epos, e, 1.0)) + 1.0
  kept = jnp.clip(kept, 0.0, cnts)
  kept = jnp.where(pmask & epos, kept, 0.0)
  score = 1.0 / jnp.sum(kept * e, axis=1, keepdims=True)

  score_ref[0] = jnp.broadcast_to(score, (_RPS, 128))
  tok = jnp.zeros((_RPS, 128), jnp.int32)
  for j in range(_RPS):
    tok = jnp.where(sub8 == j, amaxs[j], tok)
  samp_ref[0] = tok


def kernel(input_logits, cu_seqlens_q, relative_idx, batch_offsets,
           cu_filtered, temperatures, num_transfer, top_k, top_p):
  del batch_offsets, num_transfer
  rows, v = input_logits.shape
  t_total = relative_idx.shape[0]
  nb = cu_filtered.shape[0] - 1
  width = v // 8
  width_pad = ((width + 127) // 128) * 128
  nsteps = t_total // _RPS

  counts = jnp.diff(cu_filtered)
  group_ids = jnp.repeat(jnp.arange(nb), counts, total_repeat_length=t_total)
  global_rows = (jnp.take(cu_seqlens_q[:-1], group_ids, axis=0)
                 + relative_idx).astype(jnp.int32)

  logits3 = input_logits.reshape(rows, 8, width)
  logits3 = jnp.pad(logits3, ((0, 0), (0, 0), (0, width_pad - width)),
                    constant_values=_NEG)

  def in_map(j):
    return lambda i, rows_r, temps_r, ks_r, ps_r: (rows_r[i * _RPS + j], 0, 0)

  out_map = lambda i, rows_r, temps_r, ks_r, ps_r: (i, 0, 0)

  grid_spec = pltpu.PrefetchScalarGridSpec(
      num_scalar_prefetch=4,
      grid=(nsteps,),
      in_specs=[pl.BlockSpec((1, 8, width_pad), in_map(j))
                for j in range(_RPS)],
      out_specs=[
          pl.BlockSpec((1, _RPS, 128), out_map),
          pl.BlockSpec((1, _RPS, 128), out_map),
      ],
  )

  score3, samp3 = pl.pallas_call(
      functools.partial(_sampler_kernel, width=width),
      grid_spec=grid_spec,
      out_shape=[
          jax.ShapeDtypeStruct((nsteps, _RPS, 128), jnp.float32),
          jax.ShapeDtypeStruct((nsteps, _RPS, 128), jnp.int32),
      ],
  )(global_rows, temperatures, top_k, top_p, *([logits3] * _RPS))

  return (samp3[:, :, 0].reshape(t_total),
          score3[:, :, 0].reshape(t_total))
